# initial kernel scaffold (unmeasured)
import jax
import jax.numpy as jnp
from jax import lax
from jax.experimental import pallas as pl
from jax.experimental.pallas import tpu as pltpu

N_DEV = 4
M, K_SH, N = 4096, 1024, 8192
MC = M // N_DEV
TM = 128


def kernel(x, w_mat):
    def body(x_ref, w_ref, out_ref, partial_ref,
             acc, bufb, send_sems, recv_sems, cp_sems):
        i = lax.axis_index("i")
        left = jnp.mod(i - 1, N_DEV)
        right = jnp.mod(i + 1, N_DEV)

        barrier = pltpu.get_barrier_semaphore()
        for nbr in (left, right):
            pl.semaphore_signal(
                barrier, inc=1,
                device_id=(nbr,), device_id_type=pl.DeviceIdType.MESH,
            )
        pl.semaphore_wait(barrier, 2)

        for mt in range(M // TM):
            acc[...] = jnp.dot(
                x_ref[pl.ds(mt * TM, TM), :], w_ref[...],
                preferred_element_type=jnp.float32,
            )
            cp = pltpu.make_async_copy(
                acc, partial_ref.at[pl.ds(mt * TM, TM), :], cp_sems.at[0])
            cp.start()
            cp.wait()

        for s in range(N_DEV - 1):
            sc = jnp.mod(i - 1 - s, N_DEV)
            rc = jnp.mod(i - 2 - s, N_DEV)
            src_buf = partial_ref if s == 0 else out_ref
            rdma = pltpu.make_async_remote_copy(
                src_ref=src_buf.at[pl.ds(sc * MC, MC), :],
                dst_ref=out_ref.at[pl.ds(sc * MC, MC), :],
                send_sem=send_sems.at[s],
                recv_sem=recv_sems.at[s],
                device_id=(right,),
                device_id_type=pl.DeviceIdType.MESH,
            )
            rdma.start()
            rdma.wait()
            for t in range(MC // TM):
                row = rc * MC + t * TM
                ld_a = pltpu.make_async_copy(
                    partial_ref.at[pl.ds(row, TM), :], acc, cp_sems.at[0])
                ld_b = pltpu.make_async_copy(
                    out_ref.at[pl.ds(row, TM), :], bufb, cp_sems.at[1])
                ld_a.start()
                ld_b.start()
                ld_a.wait()
                ld_b.wait()
                val = acc[...] + bufb[...]
                if s == N_DEV - 2:
                    val = jnp.maximum(val, 0.0)
                acc[...] = val
                st = pltpu.make_async_copy(
                    acc, out_ref.at[pl.ds(row, TM), :], cp_sems.at[0])
                st.start()
                st.wait()

        for s in range(N_DEV - 1):
            k = (N_DEV - 1) + s
            sc = jnp.mod(i - s, N_DEV)
            rdma = pltpu.make_async_remote_copy(
                src_ref=out_ref.at[pl.ds(sc * MC, MC), :],
                dst_ref=out_ref.at[pl.ds(sc * MC, MC), :],
                send_sem=send_sems.at[k],
                recv_sem=recv_sems.at[k],
                device_id=(right,),
                device_id_type=pl.DeviceIdType.MESH,
            )
            rdma.start()
            rdma.wait()

    out, _partial = pl.pallas_call(
        body,
        out_shape=[
            jax.ShapeDtypeStruct((M, N), jnp.float32),
            jax.ShapeDtypeStruct((M, N), jnp.float32),
        ],
        in_specs=[
            pl.BlockSpec(memory_space=pltpu.VMEM),
            pl.BlockSpec(memory_space=pltpu.VMEM),
        ],
        out_specs=[
            pl.BlockSpec(memory_space=pltpu.ANY),
            pl.BlockSpec(memory_space=pltpu.ANY),
        ],
        scratch_shapes=[
            pltpu.VMEM((TM, N), jnp.float32),
            pltpu.VMEM((TM, N), jnp.float32),
            pltpu.SemaphoreType.DMA((2 * (N_DEV - 1),)),
            pltpu.SemaphoreType.DMA((2 * (N_DEV - 1),)),
            pltpu.SemaphoreType.DMA((2,)),
        ],
        compiler_params=pltpu.CompilerParams(collective_id=0),
    )(x, w_mat)
    return out


# baseline (device time: 2534953 ns/iter reference)
import jax
import jax.numpy as jnp
from jax import lax
from jax.experimental import pallas as pl
from jax.experimental.pallas import tpu as pltpu

N_DEV = 4
M, K_SH, N = 4096, 1024, 8192
MC = M // N_DEV
TM = 256


def kernel(x, w_mat):
    def body(x_ref, w_ref, out_ref, partial_ref,
             xt, acc, bufb, send_sems, recv_sems, cp_sems):
        i = lax.axis_index("i")
        left = jnp.mod(i - 1, N_DEV)
        right = jnp.mod(i + 1, N_DEV)

        barrier = pltpu.get_barrier_semaphore()
        for nbr in (left, right):
            pl.semaphore_signal(
                barrier, inc=1,
                device_id=(nbr,), device_id_type=pl.DeviceIdType.MESH,
            )
        pl.semaphore_wait(barrier, 2)

        def gemm_step(mt, _):
            row = mt * TM
            ld = pltpu.make_async_copy(
                x_ref.at[pl.ds(row, TM), :], xt, cp_sems.at[0])
            ld.start()
            ld.wait()
            acc[...] = jnp.dot(
                xt[...], w_ref[...], preferred_element_type=jnp.float32)
            st = pltpu.make_async_copy(
                acc, partial_ref.at[pl.ds(row, TM), :], cp_sems.at[0])
            st.start()
            st.wait()
            return 0

        lax.fori_loop(0, M // TM, gemm_step, 0)

        for s in range(N_DEV - 1):
            sc = jnp.mod(i - 1 - s, N_DEV)
            rc = jnp.mod(i - 2 - s, N_DEV)
            src_buf = partial_ref if s == 0 else out_ref
            rdma = pltpu.make_async_remote_copy(
                src_ref=src_buf.at[pl.ds(sc * MC, MC), :],
                dst_ref=out_ref.at[pl.ds(sc * MC, MC), :],
                send_sem=send_sems.at[s],
                recv_sem=recv_sems.at[s],
                device_id=(right,),
                device_id_type=pl.DeviceIdType.MESH,
            )
            rdma.start()
            rdma.wait()

            def add_step(t, _, rc=rc, do_relu=(s == N_DEV - 2)):
                row = rc * MC + t * TM
                ld_a = pltpu.make_async_copy(
                    partial_ref.at[pl.ds(row, TM), :], acc, cp_sems.at[0])
                ld_b = pltpu.make_async_copy(
                    out_ref.at[pl.ds(row, TM), :], bufb, cp_sems.at[1])
                ld_a.start()
                ld_b.start()
                ld_a.wait()
                ld_b.wait()
                val = acc[...] + bufb[...]
                if do_relu:
                    val = jnp.maximum(val, 0.0)
                acc[...] = val
                st = pltpu.make_async_copy(
                    acc, out_ref.at[pl.ds(row, TM), :], cp_sems.at[0])
                st.start()
                st.wait()
                return 0

            lax.fori_loop(0, MC // TM, add_step, 0)

        for s in range(N_DEV - 1):
            k = (N_DEV - 1) + s
            sc = jnp.mod(i - s, N_DEV)
            rdma = pltpu.make_async_remote_copy(
                src_ref=out_ref.at[pl.ds(sc * MC, MC), :],
                dst_ref=out_ref.at[pl.ds(sc * MC, MC), :],
                send_sem=send_sems.at[k],
                recv_sem=recv_sems.at[k],
                device_id=(right,),
                device_id_type=pl.DeviceIdType.MESH,
            )
            rdma.start()
            rdma.wait()

    out, _partial = pl.pallas_call(
        body,
        out_shape=[
            jax.ShapeDtypeStruct((M, N), jnp.float32),
            jax.ShapeDtypeStruct((M, N), jnp.float32),
        ],
        in_specs=[
            pl.BlockSpec(memory_space=pl.ANY),
            pl.BlockSpec(memory_space=pltpu.VMEM),
        ],
        out_specs=[
            pl.BlockSpec(memory_space=pl.ANY),
            pl.BlockSpec(memory_space=pl.ANY),
        ],
        scratch_shapes=[
            pltpu.VMEM((TM, K_SH), jnp.float32),
            pltpu.VMEM((TM, N), jnp.float32),
            pltpu.VMEM((TM, N), jnp.float32),
            pltpu.SemaphoreType.DMA((2 * (N_DEV - 1),)),
            pltpu.SemaphoreType.DMA((2 * (N_DEV - 1),)),
            pltpu.SemaphoreType.DMA((2,)),
        ],
        compiler_params=pltpu.CompilerParams(
            collective_id=0,
            vmem_limit_bytes=60 * 1024 * 1024,
        ),
    )(x, w_mat)
    return out


# device time: 1381611 ns/iter; 1.8348x vs baseline; 1.8348x over previous
import jax
import jax.numpy as jnp
from jax import lax
from jax.experimental import pallas as pl
from jax.experimental.pallas import tpu as pltpu

N_DEV = 4
M, K_SH, N = 4096, 1024, 8192
MC = M // N_DEV
HN = N // 2
TMG = 256
TMA = 512


def kernel(x, w_mat):
    def body(x_ref, w_ref, out_ref, partial_ref,
             xt, gacc, aacc, abuf,
             sendA, recvA, sendB, recvB, cp_sems):
        i = lax.axis_index("i")
        left = jnp.mod(i - 1, N_DEV)
        right = jnp.mod(i + 1, N_DEV)

        barrier = pltpu.get_barrier_semaphore()
        for nbr in (left, right):
            pl.semaphore_signal(
                barrier, inc=1,
                device_id=(nbr,), device_id_type=pl.DeviceIdType.MESH,
            )
        pl.semaphore_wait(barrier, 2)

        def gemm_chunk(c):
            def step(t, _):
                row = c * MC + t * TMG
                ld = pltpu.make_async_copy(
                    x_ref.at[pl.ds(row, TMG), :], xt, cp_sems.at[0])
                ld.start()
                ld.wait()
                gacc[...] = jnp.dot(
                    xt[...], w_ref[...], preferred_element_type=jnp.float32)
                st = pltpu.make_async_copy(
                    gacc, partial_ref.at[pl.ds(row, TMG), :], cp_sems.at[0])
                st.start()
                st.wait()
                return 0

            lax.fori_loop(0, MC // TMG, step, 0)

        def ring_rdma(src_buf, c, col0, ssem, rsem, slot, dev):
            return pltpu.make_async_remote_copy(
                src_ref=src_buf.at[pl.ds(c * MC, MC), pl.ds(col0, HN)],
                dst_ref=out_ref.at[pl.ds(c * MC, MC), pl.ds(col0, HN)],
                send_sem=ssem.at[slot],
                recv_sem=rsem.at[slot],
                device_id=(dev,),
                device_id_type=pl.DeviceIdType.MESH,
            )

        def add_pass(rc, col0, do_relu):
            def step(t, _):
                row = rc * MC + t * TMA
                ld_a = pltpu.make_async_copy(
                    partial_ref.at[pl.ds(row, TMA), pl.ds(col0, HN)],
                    aacc, cp_sems.at[0])
                ld_b = pltpu.make_async_copy(
                    out_ref.at[pl.ds(row, TMA), pl.ds(col0, HN)],
                    abuf, cp_sems.at[1])
                ld_a.start()
                ld_b.start()
                ld_a.wait()
                ld_b.wait()
                val = aacc[...] + abuf[...]
                if do_relu:
                    val = jnp.maximum(val, 0.0)
                aacc[...] = val
                st = pltpu.make_async_copy(
                    aacc, out_ref.at[pl.ds(row, TMA), pl.ds(col0, HN)],
                    cp_sems.at[0])
                st.start()
                st.wait()
                return 0

            lax.fori_loop(0, MC // TMA, step, 0)

        gemm_chunk(left)
        gemm_chunk(right)

        rdma_a = ring_rdma(partial_ref, left, 0, sendA, recvA, 0, right)
        rdma_b = ring_rdma(partial_ref, right, HN, sendB, recvB, 0, left)
        rdma_a.start()
        rdma_b.start()

        gemm_chunk(jnp.mod(i + 2, N_DEV))
        gemm_chunk(i)

        for s in range(N_DEV - 1):
            if s > 0:
                rdma_a = ring_rdma(
                    out_ref, jnp.mod(i - 1 - s, N_DEV), 0,
                    sendA, recvA, s, right)
                rdma_b = ring_rdma(
                    out_ref, jnp.mod(i + 1 + s, N_DEV), HN,
                    sendB, recvB, s, left)
                rdma_a.start()
                rdma_b.start()
            rdma_a.wait()
            rdma_b.wait()
            do_relu = (s == N_DEV - 2)
            add_pass(jnp.mod(i - 2 - s, N_DEV), 0, do_relu)
            add_pass(jnp.mod(i + 2 + s, N_DEV), HN, do_relu)

        for s in range(N_DEV - 1):
            k = (N_DEV - 1) + s
            rdma_a = ring_rdma(
                out_ref, jnp.mod(i - s, N_DEV), 0, sendA, recvA, k, right)
            rdma_b = ring_rdma(
                out_ref, jnp.mod(i + s, N_DEV), HN, sendB, recvB, k, left)
            rdma_a.start()
            rdma_b.start()
            rdma_a.wait()
            rdma_b.wait()

    out, _partial = pl.pallas_call(
        body,
        out_shape=[
            jax.ShapeDtypeStruct((M, N), jnp.float32),
            jax.ShapeDtypeStruct((M, N), jnp.float32),
        ],
        in_specs=[
            pl.BlockSpec(memory_space=pl.ANY),
            pl.BlockSpec(memory_space=pltpu.VMEM),
        ],
        out_specs=[
            pl.BlockSpec(memory_space=pl.ANY),
            pl.BlockSpec(memory_space=pl.ANY),
        ],
        scratch_shapes=[
            pltpu.VMEM((TMG, K_SH), jnp.float32),
            pltpu.VMEM((TMG, N), jnp.float32),
            pltpu.VMEM((TMA, HN), jnp.float32),
            pltpu.VMEM((TMA, HN), jnp.float32),
            pltpu.SemaphoreType.DMA((2 * (N_DEV - 1),)),
            pltpu.SemaphoreType.DMA((2 * (N_DEV - 1),)),
            pltpu.SemaphoreType.DMA((2 * (N_DEV - 1),)),
            pltpu.SemaphoreType.DMA((2 * (N_DEV - 1),)),
            pltpu.SemaphoreType.DMA((2,)),
        ],
        compiler_params=pltpu.CompilerParams(
            collective_id=0,
            vmem_limit_bytes=62 * 1024 * 1024,
        ),
    )(x, w_mat)
    return out


# device time: 1226450 ns/iter; 2.0669x vs baseline; 1.1265x over previous
import jax
import jax.numpy as jnp
from jax import lax
from jax.experimental import pallas as pl
from jax.experimental.pallas import tpu as pltpu

N_DEV = 4
M, K_SH, N = 4096, 1024, 8192
MC = M // N_DEV
HN = N // 2
SUB = 2
SM = MC // SUB
TMG = 256
N_SEM = 2 * (N_DEV - 1) * SUB


def kernel(x, w_mat):
    def body(x_ref, w_ref, out_ref, partial_ref,
             xt, gacc, aacc, abuf,
             sendA, recvA, sendB, recvB, cp_sems):
        i = lax.axis_index("i")
        left = jnp.mod(i - 1, N_DEV)
        right = jnp.mod(i + 1, N_DEV)

        barrier = pltpu.get_barrier_semaphore()
        for nbr in (left, right):
            pl.semaphore_signal(
                barrier, inc=1,
                device_id=(nbr,), device_id_type=pl.DeviceIdType.MESH,
            )
        pl.semaphore_wait(barrier, 2)

        def gemm_half(c, col0):
            def step(t, _):
                row = c * MC + t * TMG
                ld = pltpu.make_async_copy(
                    x_ref.at[pl.ds(row, TMG), :], xt, cp_sems.at[0])
                ld.start()
                ld.wait()
                gacc[...] = jnp.dot(
                    xt[...], w_ref[:, col0:col0 + HN],
                    preferred_element_type=jnp.float32)
                st = pltpu.make_async_copy(
                    gacc, partial_ref.at[pl.ds(row, TMG), pl.ds(col0, HN)],
                    cp_sems.at[0])
                st.start()
                st.wait()
                return 0

            lax.fori_loop(0, MC // TMG, step, 0)

        def sub_rdma(src_buf, c, col0, u, slot, ssem, rsem, dev):
            rows = c * MC + u * SM
            return pltpu.make_async_remote_copy(
                src_ref=src_buf.at[pl.ds(rows, SM), pl.ds(col0, HN)],
                dst_ref=out_ref.at[pl.ds(rows, SM), pl.ds(col0, HN)],
                send_sem=ssem.at[slot],
                recv_sem=rsem.at[slot],
                device_id=(dev,),
                device_id_type=pl.DeviceIdType.MESH,
            )

        def add_sub(rc, col0, u, do_relu):
            row = rc * MC + u * SM
            ld_a = pltpu.make_async_copy(
                partial_ref.at[pl.ds(row, SM), pl.ds(col0, HN)],
                aacc, cp_sems.at[0])
            ld_b = pltpu.make_async_copy(
                out_ref.at[pl.ds(row, SM), pl.ds(col0, HN)],
                abuf, cp_sems.at[1])
            ld_a.start()
            ld_b.start()
            ld_a.wait()
            ld_b.wait()
            val = aacc[...] + abuf[...]
            if do_relu:
                val = jnp.maximum(val, 0.0)
            aacc[...] = val
            st = pltpu.make_async_copy(
                aacc, out_ref.at[pl.ds(row, SM), pl.ds(col0, HN)],
                cp_sems.at[0])
            st.start()
            st.wait()

        rd = {}

        gemm_half(left, 0)
        for u in range(SUB):
            rd["A", 0, u] = sub_rdma(partial_ref, left, 0, u, u,
                                     sendA, recvA, right)
            rd["A", 0, u].start()
        gemm_half(right, HN)
        for u in range(SUB):
            rd["B", 0, u] = sub_rdma(partial_ref, right, HN, u, u,
                                     sendB, recvB, left)
            rd["B", 0, u].start()

        diag = jnp.mod(i + 2, N_DEV)
        gemm_half(diag, 0)
        gemm_half(diag, HN)
        gemm_half(right, 0)
        gemm_half(left, HN)
        gemm_half(i, 0)
        gemm_half(i, HN)

        for s in range(N_DEV - 1):
            rcA = jnp.mod(i - 2 - s, N_DEV)
            rcB = jnp.mod(i + 2 + s, N_DEV)
            do_relu = (s == N_DEV - 2)
            for u in range(SUB):
                rd["A", s, u].wait()
                add_sub(rcA, 0, u, do_relu)
                rd["B", s, u].wait()
                add_sub(rcB, HN, u, do_relu)
                if s < N_DEV - 2:
                    slot = (s + 1) * SUB + u
                    rd["A", s + 1, u] = sub_rdma(
                        out_ref, rcA, 0, u, slot, sendA, recvA, right)
                    rd["B", s + 1, u] = sub_rdma(
                        out_ref, rcB, HN, u, slot, sendB, recvB, left)
                else:
                    slot = (N_DEV - 1) * SUB + u
                    rd["GA", 0, u] = sub_rdma(
                        out_ref, i, 0, u, slot, sendA, recvA, right)
                    rd["GB", 0, u] = sub_rdma(
                        out_ref, i, HN, u, slot, sendB, recvB, left)
                    rd["GA", 0, u].start()
                    rd["GB", 0, u].start()
                    continue
                rd["A", s + 1, u].start()
                rd["B", s + 1, u].start()

        for s in range(N_DEV - 1):
            for u in range(SUB):
                rd["GA", s, u].wait()
                rd["GB", s, u].wait()
                if s < N_DEV - 2:
                    slot = (N_DEV - 1 + s + 1) * SUB + u
                    rd["GA", s + 1, u] = sub_rdma(
                        out_ref, jnp.mod(i - 1 - s, N_DEV), 0, u, slot,
                        sendA, recvA, right)
                    rd["GB", s + 1, u] = sub_rdma(
                        out_ref, jnp.mod(i + 1 + s, N_DEV), HN, u, slot,
                        sendB, recvB, left)
                    rd["GA", s + 1, u].start()
                    rd["GB", s + 1, u].start()

    out, _partial = pl.pallas_call(
        body,
        out_shape=[
            jax.ShapeDtypeStruct((M, N), jnp.float32),
            jax.ShapeDtypeStruct((M, N), jnp.float32),
        ],
        in_specs=[
            pl.BlockSpec(memory_space=pl.ANY),
            pl.BlockSpec(memory_space=pltpu.VMEM),
        ],
        out_specs=[
            pl.BlockSpec(memory_space=pl.ANY),
            pl.BlockSpec(memory_space=pl.ANY),
        ],
        scratch_shapes=[
            pltpu.VMEM((TMG, K_SH), jnp.float32),
            pltpu.VMEM((TMG, HN), jnp.float32),
            pltpu.VMEM((SM, HN), jnp.float32),
            pltpu.VMEM((SM, HN), jnp.float32),
            pltpu.SemaphoreType.DMA((N_SEM,)),
            pltpu.SemaphoreType.DMA((N_SEM,)),
            pltpu.SemaphoreType.DMA((N_SEM,)),
            pltpu.SemaphoreType.DMA((N_SEM,)),
            pltpu.SemaphoreType.DMA((2,)),
        ],
        compiler_params=pltpu.CompilerParams(
            collective_id=0,
            vmem_limit_bytes=62 * 1024 * 1024,
        ),
    )(x, w_mat)
    return out


# device time: 1203342 ns/iter; 2.1066x vs baseline; 1.0192x over previous
import jax
import jax.numpy as jnp
from jax import lax
from jax.experimental import pallas as pl
from jax.experimental.pallas import tpu as pltpu

N_DEV = 4
M, K_SH, N = 4096, 1024, 8192
MC = M // N_DEV
HN = N // 2
SUB = 2
SM = MC // SUB
TMG = 256
N_SEM = 2 * (N_DEV - 1) * SUB


def kernel(x, w_mat):
    def body(x_ref, w_ref, out_ref, partial_ref,
             xt, gacc, aacc, abuf,
             sendA, recvA, sendB, recvB, cp_sems):
        i = lax.axis_index("i")
        left = jnp.mod(i - 1, N_DEV)
        right = jnp.mod(i + 1, N_DEV)

        barrier = pltpu.get_barrier_semaphore()
        for nbr in (left, right):
            pl.semaphore_signal(
                barrier, inc=1,
                device_id=(nbr,), device_id_type=pl.DeviceIdType.MESH,
            )
        pl.semaphore_wait(barrier, 2)

        def gemm_half(c, col0):
            def step(t, _):
                row = c * MC + t * TMG
                ld = pltpu.make_async_copy(
                    x_ref.at[pl.ds(row, TMG), :], xt, cp_sems.at[0])
                ld.start()
                ld.wait()
                gacc[...] = jnp.dot(
                    xt[...], w_ref[:, col0:col0 + HN],
                    preferred_element_type=jnp.float32)
                st = pltpu.make_async_copy(
                    gacc, partial_ref.at[pl.ds(row, TMG), pl.ds(col0, HN)],
                    cp_sems.at[0])
                st.start()
                st.wait()
                return 0

            lax.fori_loop(0, MC // TMG, step, 0)

        def gemm_sub(c, col0, u):
            def step(t, _):
                row = c * MC + u * SM + t * TMG
                ld = pltpu.make_async_copy(
                    x_ref.at[pl.ds(row, TMG), :], xt, cp_sems.at[0])
                ld.start()
                ld.wait()
                gacc[...] = jnp.dot(
                    xt[...], w_ref[:, col0:col0 + HN],
                    preferred_element_type=jnp.float32)
                st = pltpu.make_async_copy(
                    gacc, partial_ref.at[pl.ds(row, TMG), pl.ds(col0, HN)],
                    cp_sems.at[0])
                st.start()
                st.wait()
                return 0

            lax.fori_loop(0, SM // TMG, step, 0)

        def sub_rdma(src_buf, c, col0, u, slot, ssem, rsem, dev):
            rows = c * MC + u * SM
            return pltpu.make_async_remote_copy(
                src_ref=src_buf.at[pl.ds(rows, SM), pl.ds(col0, HN)],
                dst_ref=out_ref.at[pl.ds(rows, SM), pl.ds(col0, HN)],
                send_sem=ssem.at[slot],
                recv_sem=rsem.at[slot],
                device_id=(dev,),
                device_id_type=pl.DeviceIdType.MESH,
            )

        def add_sub(rc, col0, u, do_relu):
            row = rc * MC + u * SM
            ld_a = pltpu.make_async_copy(
                partial_ref.at[pl.ds(row, SM), pl.ds(col0, HN)],
                aacc, cp_sems.at[0])
            ld_b = pltpu.make_async_copy(
                out_ref.at[pl.ds(row, SM), pl.ds(col0, HN)],
                abuf, cp_sems.at[1])
            ld_a.start()
            ld_b.start()
            ld_a.wait()
            ld_b.wait()
            val = aacc[...] + abuf[...]
            if do_relu:
                val = jnp.maximum(val, 0.0)
            aacc[...] = val
            st = pltpu.make_async_copy(
                aacc, out_ref.at[pl.ds(row, SM), pl.ds(col0, HN)],
                cp_sems.at[0])
            st.start()
            st.wait()

        rd = {}

        diag = jnp.mod(i + 2, N_DEV)
        for u in range(SUB):
            gemm_sub(left, 0, u)
            rd["A", 0, u] = sub_rdma(partial_ref, left, 0, u, u,
                                     sendA, recvA, right)
            rd["A", 0, u].start()
            gemm_sub(right, HN, u)
            rd["B", 0, u] = sub_rdma(partial_ref, right, HN, u, u,
                                     sendB, recvB, left)
            rd["B", 0, u].start()

        pending_gemm = [
            (diag, 0),
            (diag, HN),
            (right, 0),
            (left, HN),
            (i, 0),
            (i, HN),
        ]
        gemm_half(*pending_gemm.pop(0))
        gemm_half(*pending_gemm.pop(0))

        for s in range(N_DEV - 1):
            rcA = jnp.mod(i - 2 - s, N_DEV)
            rcB = jnp.mod(i + 2 + s, N_DEV)
            do_relu = (s == N_DEV - 2)
            for u in range(SUB):
                rd["A", s, u].wait()
                add_sub(rcA, 0, u, do_relu)
                rd["B", s, u].wait()
                add_sub(rcB, HN, u, do_relu)
                if s < N_DEV - 2:
                    slot = (s + 1) * SUB + u
                    rd["A", s + 1, u] = sub_rdma(
                        out_ref, rcA, 0, u, slot, sendA, recvA, right)
                    rd["B", s + 1, u] = sub_rdma(
                        out_ref, rcB, HN, u, slot, sendB, recvB, left)
                else:
                    slot = (N_DEV - 1) * SUB + u
                    rd["GA", 0, u] = sub_rdma(
                        out_ref, i, 0, u, slot, sendA, recvA, right)
                    rd["GB", 0, u] = sub_rdma(
                        out_ref, i, HN, u, slot, sendB, recvB, left)
                    rd["GA", 0, u].start()
                    rd["GB", 0, u].start()
                    continue
                rd["A", s + 1, u].start()
                rd["B", s + 1, u].start()
                if pending_gemm:
                    gemm_half(*pending_gemm.pop(0))

        for s in range(N_DEV - 1):
            for u in range(SUB):
                rd["GA", s, u].wait()
                rd["GB", s, u].wait()
                if s < N_DEV - 2:
                    slot = (N_DEV - 1 + s + 1) * SUB + u
                    rd["GA", s + 1, u] = sub_rdma(
                        out_ref, jnp.mod(i - 1 - s, N_DEV), 0, u, slot,
                        sendA, recvA, right)
                    rd["GB", s + 1, u] = sub_rdma(
                        out_ref, jnp.mod(i + 1 + s, N_DEV), HN, u, slot,
                        sendB, recvB, left)
                    rd["GA", s + 1, u].start()
                    rd["GB", s + 1, u].start()

    out, _partial = pl.pallas_call(
        body,
        out_shape=[
            jax.ShapeDtypeStruct((M, N), jnp.float32),
            jax.ShapeDtypeStruct((M, N), jnp.float32),
        ],
        in_specs=[
            pl.BlockSpec(memory_space=pl.ANY),
            pl.BlockSpec(memory_space=pltpu.VMEM),
        ],
        out_specs=[
            pl.BlockSpec(memory_space=pl.ANY),
            pl.BlockSpec(memory_space=pl.ANY),
        ],
        scratch_shapes=[
            pltpu.VMEM((TMG, K_SH), jnp.float32),
            pltpu.VMEM((TMG, HN), jnp.float32),
            pltpu.VMEM((SM, HN), jnp.float32),
            pltpu.VMEM((SM, HN), jnp.float32),
            pltpu.SemaphoreType.DMA((N_SEM,)),
            pltpu.SemaphoreType.DMA((N_SEM,)),
            pltpu.SemaphoreType.DMA((N_SEM,)),
            pltpu.SemaphoreType.DMA((N_SEM,)),
            pltpu.SemaphoreType.DMA((2,)),
        ],
        compiler_params=pltpu.CompilerParams(
            collective_id=0,
            vmem_limit_bytes=62 * 1024 * 1024,
        ),
    )(x, w_mat)
    return out


# device time: 1198989 ns/iter; 2.1142x vs baseline; 1.0036x over previous
import os

import jax
import jax.numpy as jnp
from jax import lax
from jax.experimental import pallas as pl
from jax.experimental.pallas import tpu as pltpu

N_DEV = 4
M, K_SH, N = 4096, 1024, 8192
MC = M // N_DEV
RH = MC // 2
SUB = 2
SM = RH // SUB
TMG = SM
N_SEM = 2 * (N_DEV - 1) * SUB

_SKIP_GEMM = bool(os.environ.get("KERNEL_SKIP_GEMM"))
_SKIP_ADD = bool(os.environ.get("KERNEL_SKIP_ADD"))


def kernel(x, w_mat):
    def body(x_ref, w_ref, out_ref, partial_ref,
             xt, gacc, aacc, abuf,
             sendA, recvA, sendB, recvB, cp_sems):
        i = lax.axis_index("i")
        left = jnp.mod(i - 1, N_DEV)
        right = jnp.mod(i + 1, N_DEV)
        diag = jnp.mod(i + 2, N_DEV)

        barrier = pltpu.get_barrier_semaphore()
        for nbr in (left, right):
            pl.semaphore_signal(
                barrier, inc=1,
                device_id=(nbr,), device_id_type=pl.DeviceIdType.MESH,
            )
        pl.semaphore_wait(barrier, 2)

        def gemm_tile(c, t):
            if _SKIP_GEMM:
                return
            row = c * MC + t * TMG
            ld = pltpu.make_async_copy(
                x_ref.at[pl.ds(row, TMG), :], xt, cp_sems.at[0])
            ld.start()
            ld.wait()
            gacc[...] = jnp.dot(
                xt[...], w_ref[...], preferred_element_type=jnp.float32)
            st = pltpu.make_async_copy(
                gacc, partial_ref.at[pl.ds(row, TMG), :], cp_sems.at[0])
            st.start()
            st.wait()

        def gemm_range(c, t0, nt):
            if _SKIP_GEMM:
                return
            lax.fori_loop(t0, t0 + nt, lambda t, _: (gemm_tile(c, t), 0)[1], 0)

        def sub_row(c, roff, u):
            return c * MC + roff + u * SM

        def sub_rdma(src_buf, c, roff, u, slot, ssem, rsem, dev):
            row = sub_row(c, roff, u)
            return pltpu.make_async_remote_copy(
                src_ref=src_buf.at[pl.ds(row, SM), :],
                dst_ref=out_ref.at[pl.ds(row, SM), :],
                send_sem=ssem.at[slot],
                recv_sem=rsem.at[slot],
                device_id=(dev,),
                device_id_type=pl.DeviceIdType.MESH,
            )

        def add_sub(rc, roff, u, do_relu):
            if _SKIP_ADD:
                return
            row = sub_row(rc, roff, u)
            ld_a = pltpu.make_async_copy(
                partial_ref.at[pl.ds(row, SM), :], aacc, cp_sems.at[0])
            ld_b = pltpu.make_async_copy(
                out_ref.at[pl.ds(row, SM), :], abuf, cp_sems.at[1])
            ld_a.start()
            ld_b.start()
            ld_a.wait()
            ld_b.wait()
            val = aacc[...] + abuf[...]
            if do_relu:
                val = jnp.maximum(val, 0.0)
            aacc[...] = val
            st = pltpu.make_async_copy(
                aacc, out_ref.at[pl.ds(row, SM), :], cp_sems.at[0])
            st.start()
            st.wait()

        rd = {}

        for u in range(SUB):
            gemm_tile(left, u)
            rd["A", 0, u] = sub_rdma(partial_ref, left, 0, u, u,
                                     sendA, recvA, right)
            rd["A", 0, u].start()
            gemm_tile(right, SUB + u)
            rd["B", 0, u] = sub_rdma(partial_ref, right, RH, u, u,
                                     sendB, recvB, left)
            rd["B", 0, u].start()

        gemm_range(diag, 0, 2 * SUB)
        pending_gemm = [
            (right, 0, SUB),
            (left, SUB, SUB),
            (i, 0, SUB),
            (i, SUB, SUB),
        ]

        for s in range(N_DEV - 1):
            rcA = jnp.mod(i - 2 - s, N_DEV)
            rcB = jnp.mod(i + 2 + s, N_DEV)
            do_relu = (s == N_DEV - 2)
            for u in range(SUB):
                rd["A", s, u].wait()
                add_sub(rcA, 0, u, do_relu)
                rd["B", s, u].wait()
                add_sub(rcB, RH, u, do_relu)
                if s < N_DEV - 2:
                    slot = (s + 1) * SUB + u
                    rd["A", s + 1, u] = sub_rdma(
                        out_ref, rcA, 0, u, slot, sendA, recvA, right)
                    rd["B", s + 1, u] = sub_rdma(
                        out_ref, rcB, RH, u, slot, sendB, recvB, left)
                else:
                    slot = (N_DEV - 1) * SUB + u
                    rd["GA", 0, u] = sub_rdma(
                        out_ref, i, 0, u, slot, sendA, recvA, right)
                    rd["GB", 0, u] = sub_rdma(
                        out_ref, i, RH, u, slot, sendB, recvB, left)
                    rd["GA", 0, u].start()
                    rd["GB", 0, u].start()
                    continue
                rd["A", s + 1, u].start()
                rd["B", s + 1, u].start()
                if pending_gemm:
                    gemm_range(*pending_gemm.pop(0))

        for s in range(N_DEV - 1):
            for u in range(SUB):
                rd["GA", s, u].wait()
                rd["GB", s, u].wait()
                if s < N_DEV - 2:
                    slot = (N_DEV - 1 + s + 1) * SUB + u
                    rd["GA", s + 1, u] = sub_rdma(
                        out_ref, jnp.mod(i - 1 - s, N_DEV), 0, u, slot,
                        sendA, recvA, right)
                    rd["GB", s + 1, u] = sub_rdma(
                        out_ref, jnp.mod(i + 1 + s, N_DEV), RH, u, slot,
                        sendB, recvB, left)
                    rd["GA", s + 1, u].start()
                    rd["GB", s + 1, u].start()

    out, _partial = pl.pallas_call(
        body,
        out_shape=[
            jax.ShapeDtypeStruct((M, N), jnp.float32),
            jax.ShapeDtypeStruct((M, N), jnp.float32),
        ],
        in_specs=[
            pl.BlockSpec(memory_space=pl.ANY),
            pl.BlockSpec(memory_space=pltpu.VMEM),
        ],
        out_specs=[
            pl.BlockSpec(memory_space=pl.ANY),
            pl.BlockSpec(memory_space=pl.ANY),
        ],
        scratch_shapes=[
            pltpu.VMEM((TMG, K_SH), jnp.float32),
            pltpu.VMEM((TMG, N), jnp.float32),
            pltpu.VMEM((SM, N), jnp.float32),
            pltpu.VMEM((SM, N), jnp.float32),
            pltpu.SemaphoreType.DMA((N_SEM,)),
            pltpu.SemaphoreType.DMA((N_SEM,)),
            pltpu.SemaphoreType.DMA((N_SEM,)),
            pltpu.SemaphoreType.DMA((N_SEM,)),
            pltpu.SemaphoreType.DMA((2,)),
        ],
        compiler_params=pltpu.CompilerParams(
            collective_id=0,
            vmem_limit_bytes=62 * 1024 * 1024,
        ),
    )(x, w_mat)
    return out
